# baseline (speedup 1.0000x reference)
import functools

import jax
import jax.numpy as jnp
from jax import lax
from jax.experimental import pallas as pl
from jax.experimental.pallas import tpu as pltpu
from jax.experimental.pallas import tpu_sc as plsc

_ROWS = 24576
_LANES = 128


def _make_sc_copy():
    info = plsc.get_sparse_core_info()
    nc, ns = info.num_cores, info.num_subcores
    nw = nc * ns
    rows_w = _ROWS // nw
    half = rows_w // 2
    mesh = plsc.VectorSubcoreMesh(core_axis_name="c", subcore_axis_name="s")

    @functools.partial(
        pl.kernel,
        mesh=mesh,
        out_type=jax.ShapeDtypeStruct((_ROWS, _LANES), jnp.float32),
        scratch_types=[
            pltpu.VMEM((half, _LANES), jnp.float32),
            pltpu.VMEM((half, _LANES), jnp.float32),
            pltpu.SemaphoreType.DMA,
            pltpu.SemaphoreType.DMA,
        ],
    )
    def sc_copy(v_hbm, o_hbm, buf0, buf1, sem0, sem1):
        wid = lax.axis_index("s") * nc + lax.axis_index("c")
        base = wid * rows_w
        cp0 = pltpu.async_copy(v_hbm.at[pl.ds(base, half)], buf0, sem0)
        cp1 = pltpu.async_copy(v_hbm.at[pl.ds(base + half, half)], buf1, sem1)
        cp0.wait()
        o0 = pltpu.async_copy(buf0, o_hbm.at[pl.ds(base, half)], sem0)
        cp1.wait()
        o1 = pltpu.async_copy(buf1, o_hbm.at[pl.ds(base + half, half)], sem1)
        o0.wait()
        o1.wait()

    return sc_copy


def kernel(queries, keys, values):
    b, l, h, d = values.shape
    vt = jnp.transpose(values, (0, 2, 3, 1)).reshape(_ROWS, _LANES)
    out = _make_sc_copy()(vt)
    return jnp.transpose(out.reshape(b, h, d, l), (0, 1, 3, 2))


# speedup vs baseline: 6.2379x; 6.2379x over previous
import jax
import jax.numpy as jnp
from jax.experimental import pallas as pl
from jax.experimental.pallas import tpu as pltpu

_CHUNKS = 6
_NBUF = 6


def _dma_pipeline_body(v_ref, o_ref, buf, in_sems, out_sems):
    def in_copy(k):
        s = k % _NBUF
        return pltpu.make_async_copy(v_ref.at[k], buf.at[s], in_sems.at[s])

    def out_copy(k):
        s = k % _NBUF
        return pltpu.make_async_copy(buf.at[s], o_ref.at[k], out_sems.at[s])

    for k in range(_NBUF):
        in_copy(k).start()
    for k in range(_CHUNKS):
        in_copy(k).wait()
        out_copy(k).start()
        nxt = k + _NBUF
        if nxt < _CHUNKS:
            out_copy(k).wait()
            in_copy(nxt).start()
    for k in range(_CHUNKS - _NBUF, _CHUNKS):
        out_copy(k).wait()


def kernel(queries, keys, values):
    b, l, h, d = values.shape
    vt = jnp.transpose(values, (0, 2, 3, 1)).reshape(_CHUNKS, (b * h * d) // _CHUNKS, l)
    out = pl.pallas_call(
        _dma_pipeline_body,
        in_specs=[pl.BlockSpec(memory_space=pltpu.MemorySpace.HBM)],
        out_specs=pl.BlockSpec(memory_space=pltpu.MemorySpace.HBM),
        out_shape=jax.ShapeDtypeStruct(vt.shape, vt.dtype),
        scratch_shapes=[
            pltpu.VMEM((_NBUF,) + vt.shape[1:], vt.dtype),
            pltpu.SemaphoreType.DMA((_NBUF,)),
            pltpu.SemaphoreType.DMA((_NBUF,)),
        ],
    )(vt)
    return jnp.transpose(out.reshape(b, h, d, l), (0, 1, 3, 2))
